# Initial kernel scaffold; baseline (speedup 1.0000x reference)
import jax
import jax.numpy as jnp
from jax.experimental import pallas as pl


def kernel(length, rel_key_table, rel_value_table):
    raise NotImplementedError("write your pallas kernel here")


# speedup vs baseline: 6.5735x; 6.5735x over previous
import functools

import jax
import jax.numpy as jnp
from jax import lax
from jax.experimental import pallas as pl
from jax.experimental.pallas import tpu as pltpu
from jax.experimental.pallas import tpu_sc as plsc

_MAX_REL = 128
_HEAD = 64
_VOCAB = 2 * _MAX_REL + 1
_L = 2048
_SAT = _L - 1 - _MAX_REL

_NC = 2
_NS = 16
_NW = _NC * _NS
_ROWS_PER_W = _L // _NW
_W = 1024
_WIN = _W + _ROWS_PER_W - 1
_LANES = 16


def _rpe_body(key_hbm, val_hbm, out_k, out_v, tab_v, win_v, sem):
    wid = lax.axis_index("s") * _NC + lax.axis_index("c")
    r0 = wid * _ROWS_PER_W

    for tab_hbm, out_hbm in ((key_hbm, out_k), (val_hbm, out_v)):
        pltpu.async_copy(tab_hbm, tab_v, sem).wait()
        for j0 in (0, _W):
            p0 = (_L - _ROWS_PER_W) - r0 + j0

            def build_row(m, _, p0=p0):
                idx = jnp.clip(p0 + m - _SAT, 0, _VOCAB - 1)
                for c in range(_HEAD // _LANES):
                    win_v[pl.ds(m * _HEAD + c * _LANES, _LANES)] = tab_v[
                        pl.ds(idx * _HEAD + c * _LANES, _LANES)
                    ]
                return _

            lax.fori_loop(0, _WIN, build_row, 0, unroll=4)
            handles = [
                pltpu.async_copy(
                    win_v.at[pl.ds((_ROWS_PER_W - 1 - k) * _HEAD, _W * _HEAD)],
                    out_hbm.at[pl.ds(((r0 + k) * _L + j0) * _HEAD, _W * _HEAD)],
                    sem,
                )
                for k in range(_ROWS_PER_W)
            ]
            for h in handles:
                h.wait()
            plsc.subcore_barrier()


@jax.jit
def _rpe_call(rel_key_table, rel_value_table):
    mesh = plsc.VectorSubcoreMesh(core_axis_name="c", subcore_axis_name="s")
    fn = functools.partial(
        pl.kernel,
        mesh=mesh,
        out_type=(
            jax.ShapeDtypeStruct((_L * _L * _HEAD,), jnp.float32),
            jax.ShapeDtypeStruct((_L * _L * _HEAD,), jnp.float32),
        ),
        scratch_types=[
            pltpu.VMEM((_VOCAB * _HEAD,), jnp.float32),
            pltpu.VMEM((_WIN * _HEAD,), jnp.float32),
            pltpu.SemaphoreType.DMA,
        ],
    )(_rpe_body)
    return fn(rel_key_table, rel_value_table)


def kernel(length, rel_key_table, rel_value_table):
    out_k, out_v = _rpe_call(
        rel_key_table.reshape(_VOCAB * _HEAD), rel_value_table.reshape(_VOCAB * _HEAD)
    )
    return (
        out_k.reshape(_L, _L, _HEAD),
        out_v.reshape(_L, _L, _HEAD),
    )
